# Initial kernel scaffold; baseline (speedup 1.0000x reference)
import jax
import jax.numpy as jnp
from jax.experimental import pallas as pl


def kernel(x, adj_bias, e_dense, pool_t, c0_wl, c0_bl, c0_wr, c0_br, c0_we, c0_att, c0_bias, c1_wl, c1_bl, c1_wr, c1_br, c1_we, c1_att, c1_bias, node_lin_w, node_lin_b, graph_lin_w, graph_lin_b, node_norm_g, node_norm_b, graph_norm_g, graph_norm_b):
    raise NotImplementedError("write your pallas kernel here")


# speedup vs baseline: 11.9473x; 11.9473x over previous
import functools

import jax
import jax.numpy as jnp
from jax.experimental import pallas as pl
from jax.experimental.pallas import tpu as pltpu

NEG_SLOPE = 0.2
LN_EPS = 1e-5


def _gat_block(xin, e_flat, adj, wl, bl, wr, br, we, att, bias, *,
               m, heads, ch, apply_relu):
    xl = jnp.dot(xin, wl, preferred_element_type=jnp.float32) + bl
    xr = jnp.dot(xin, wr, preferred_element_type=jnp.float32) + br
    e_prj = jnp.dot(e_flat, we, preferred_element_type=jnp.float32)
    e_prj = e_prj.reshape(m, m, heads * ch)

    outs = []
    for h in range(heads):
        sl = slice(h * ch, (h + 1) * ch)
        s = e_prj[:, :, sl] + xr[:, None, sl] + xl[None, :, sl]
        s = jnp.where(s > 0, s, NEG_SLOPE * s)
        logits = jnp.sum(s * att[h][None, None, :], axis=-1) + adj
        mx = jnp.max(logits, axis=-1, keepdims=True)
        p = jnp.exp(logits - mx)
        alpha = p / jnp.sum(p, axis=-1, keepdims=True)
        outs.append(jnp.dot(alpha, xl[:, sl],
                            preferred_element_type=jnp.float32))
    out = jnp.concatenate(outs, axis=-1) + bias
    if apply_relu:
        out = jnp.maximum(out, 0.0)
    return out


def _encoder_block_kernel(x_ref, adj_ref, e_ref, pool_ref,
                          w0l_ref, b0l_ref, w0r_ref, b0r_ref, w0e_ref,
                          a0_ref, c0b_ref,
                          w1l_ref, b1l_ref, w1r_ref, b1r_ref, w1e_ref,
                          a1_ref, c1b_ref,
                          wn_ref, bn_ref, gn_ref, btn_ref,
                          wg_ref, bg_ref, gg_ref, btg_ref,
                          local_ref, global_ref, *, heads, ch):
    m = x_ref.shape[0]
    e_dim = e_ref.shape[-1]
    adj = adj_ref[0]
    e_flat = e_ref[...].reshape(m * m, e_dim)

    x1 = _gat_block(x_ref[...], e_flat, adj,
                    w0l_ref[...], b0l_ref[...], w0r_ref[...], b0r_ref[...],
                    w0e_ref[...], a0_ref[...], c0b_ref[...],
                    m=m, heads=heads, ch=ch, apply_relu=True)
    x2 = _gat_block(x1, e_flat, adj,
                    w1l_ref[...], b1l_ref[...], w1r_ref[...], b1r_ref[...],
                    w1e_ref[...], a1_ref[...], c1b_ref[...],
                    m=m, heads=heads, ch=ch, apply_relu=False)

    y = jnp.dot(x2, wn_ref[...], preferred_element_type=jnp.float32) + bn_ref[...]
    mu = jnp.mean(y, axis=-1, keepdims=True)
    var = jnp.mean(jnp.square(y - mu), axis=-1, keepdims=True)
    local_ref[...] = (y - mu) * jax.lax.rsqrt(var + LN_EPS) * gn_ref[...] + btn_ref[...]

    w_pool = jnp.sum(pool_ref[...], axis=1, keepdims=True)
    pooled = jax.lax.dot_general(
        w_pool, x2, dimension_numbers=(((0,), (0,)), ((), ())),
        preferred_element_type=jnp.float32)
    g = jnp.dot(pooled, wg_ref[...], preferred_element_type=jnp.float32) + bg_ref[...]
    mug = jnp.mean(g, axis=-1, keepdims=True)
    varg = jnp.mean(jnp.square(g - mug), axis=-1, keepdims=True)
    global_ref[0] = (g - mug) * jax.lax.rsqrt(varg + LN_EPS) * gg_ref[...] + btg_ref[...]


def kernel(x, adj_bias, e_dense, pool_t,
           c0_wl, c0_bl, c0_wr, c0_br, c0_we, c0_att, c0_bias,
           c1_wl, c1_bl, c1_wr, c1_br, c1_we, c1_att, c1_bias,
           node_lin_w, node_lin_b, graph_lin_w, graph_lin_b,
           node_norm_g, node_norm_b, graph_norm_g, graph_norm_b):
    n_pad, f = x.shape
    bsz = pool_t.shape[1]
    m = n_pad // bsz
    e_dim = e_dense.shape[-1]
    heads, ch = c0_att.shape
    hc = heads * ch
    c_out = node_lin_w.shape[1]

    row2 = lambda a: a.reshape(1, -1)

    idx = jnp.arange(bsz)
    adj_diag = adj_bias.reshape(bsz, m, bsz, m)[idx, :, idx, :]
    e5 = e_dense.reshape(bsz, m, bsz, m, e_dim)

    grid = (bsz,)
    local, global_ = pl.pallas_call(
        functools.partial(_encoder_block_kernel, heads=heads, ch=ch),
        grid=grid,
        in_specs=[
            pl.BlockSpec((m, f), lambda g: (g, 0)),
            pl.BlockSpec((1, m, m), lambda g: (g, 0, 0)),
            pl.BlockSpec((1, m, 1, m, e_dim),
                         lambda g: (g, 0, g, 0, 0)),
            pl.BlockSpec((m, bsz), lambda g: (g, 0)),
            pl.BlockSpec((f, hc), lambda g: (0, 0)),
            pl.BlockSpec((1, hc), lambda g: (0, 0)),
            pl.BlockSpec((f, hc), lambda g: (0, 0)),
            pl.BlockSpec((1, hc), lambda g: (0, 0)),
            pl.BlockSpec((e_dim, hc), lambda g: (0, 0)),
            pl.BlockSpec((heads, ch), lambda g: (0, 0)),
            pl.BlockSpec((1, hc), lambda g: (0, 0)),
            pl.BlockSpec((hc, hc), lambda g: (0, 0)),
            pl.BlockSpec((1, hc), lambda g: (0, 0)),
            pl.BlockSpec((hc, hc), lambda g: (0, 0)),
            pl.BlockSpec((1, hc), lambda g: (0, 0)),
            pl.BlockSpec((e_dim, hc), lambda g: (0, 0)),
            pl.BlockSpec((heads, ch), lambda g: (0, 0)),
            pl.BlockSpec((1, hc), lambda g: (0, 0)),
            pl.BlockSpec((hc, c_out), lambda g: (0, 0)),
            pl.BlockSpec((1, c_out), lambda g: (0, 0)),
            pl.BlockSpec((1, c_out), lambda g: (0, 0)),
            pl.BlockSpec((1, c_out), lambda g: (0, 0)),
            pl.BlockSpec((hc, c_out), lambda g: (0, 0)),
            pl.BlockSpec((1, c_out), lambda g: (0, 0)),
            pl.BlockSpec((1, c_out), lambda g: (0, 0)),
            pl.BlockSpec((1, c_out), lambda g: (0, 0)),
        ],
        out_specs=[
            pl.BlockSpec((m, c_out), lambda g: (g, 0)),
            pl.BlockSpec((1, 1, c_out), lambda g: (g, 0, 0)),
        ],
        out_shape=(jax.ShapeDtypeStruct((n_pad, c_out), jnp.float32),
                   jax.ShapeDtypeStruct((bsz, 1, c_out), jnp.float32)),
        compiler_params=pltpu.CompilerParams(
            dimension_semantics=("arbitrary",),
            vmem_limit_bytes=100 * 1024 * 1024),
    )(x, adj_diag, e5, pool_t,
      c0_wl, row2(c0_bl), c0_wr, row2(c0_br), c0_we, c0_att, row2(c0_bias),
      c1_wl, row2(c1_bl), c1_wr, row2(c1_br), c1_we, c1_att, row2(c1_bias),
      node_lin_w, row2(node_lin_b), row2(node_norm_g), row2(node_norm_b),
      graph_lin_w, row2(graph_lin_b), row2(graph_norm_g), row2(graph_norm_b))
    return local, global_.reshape(bsz, c_out)
